# baseline (speedup 1.0000x reference)
import math

import jax
import jax.numpy as jnp
from jax import lax
from jax.experimental import pallas as pl
from jax.experimental.pallas import tpu as pltpu

NB = 16
C = 64
HW = 1024
NE = 1024
N_TOK = NB * HW
BPS = 4
NSTEP = NB // BPS


def _quantize(x, emb, em2, e2a):
    mm2 = lax.dot_general(em2, x, (((1,), (0,)), ((), ())),
                          preferred_element_type=jnp.float32)
    xn = jnp.sum(x * x, axis=0, keepdims=True)
    xa = jnp.concatenate([jnp.ones((1, HW), jnp.float32), xn], axis=0)
    t = lax.dot_general(e2a, xa, (((1,), (0,)), ((), ())),
                        preferred_element_type=jnp.float32)
    d2 = t + mm2
    mind = jnp.min(d2, axis=0, keepdims=True)
    eqm = d2 == mind
    m = jnp.where(eqm, 1.0, 0.0)
    zq = lax.dot_general(emb, m, (((0,), (0,)), ((), ())),
                         preferred_element_type=jnp.float32)
    diff = zq - x
    zq_st = x + diff
    s = jnp.sum(diff * diff)
    h = jnp.sum(m, axis=1, keepdims=True)
    return zq_st, s, h, eqm


def _exact_onehot(eqm):
    iota = lax.broadcasted_iota(jnp.int32, eqm.shape, 0)
    idxv = jnp.min(jnp.where(eqm, iota, NE), axis=0, keepdims=True)
    return (iota == idxv).astype(jnp.float32)


def _vq_body(x_ref, e_ref, zq_ref, loss_ref, perp_ref,
             em2_ref, e2_ref, hist_ref, acc_ref):
    b = pl.program_id(0)
    emb = e_ref[...]

    @pl.when(b == 0)
    def _init():
        acc_ref[0] = 0.0
        hist_ref[...] = jnp.zeros_like(hist_ref)
        e2 = jnp.sum(emb * emb, axis=1, keepdims=True)
        e2_ref[...] = jnp.concatenate(
            [e2, jnp.ones((NE, 1), jnp.float32)], axis=1)
        em2_ref[...] = emb * (-2.0)

    em2 = em2_ref[...]
    e2a = e2_ref[...]

    res = [_quantize(x_ref[j], emb, em2, e2a) for j in range(BPS)]
    for j, (zq_st, _, _, _) in enumerate(res):
        zq_ref[j] = zq_st
    s_all = res[0][1]
    h_all = res[0][2]
    for j in range(1, BPS):
        s_all = s_all + res[j][1]
        h_all = h_all + res[j][2]
    acc_ref[0] += s_all
    hist_ref[...] += h_all

    @pl.when(jnp.sum(h_all) > BPS * HW + 0.5)
    def _fix():
        for j in range(BPS):
            _, s1, h1, eqm = res[j]
            x_j = x_ref[j]
            onehot = _exact_onehot(eqm)
            zq2 = lax.dot_general(emb, onehot, (((0,), (0,)), ((), ())),
                                  preferred_element_type=jnp.float32)
            diff2 = zq2 - x_j
            zq_ref[j] = x_j + diff2
            acc_ref[0] += jnp.sum(diff2 * diff2) - s1
            hist_ref[...] += jnp.sum(onehot, axis=1, keepdims=True) - h1

    @pl.when(b == NSTEP - 1)
    def _fini():
        loss_ref[0, 0] = 1.25 * acc_ref[0] / (NB * C * HW)
        probs = hist_ref[...] * (1.0 / N_TOK)
        ent = -jnp.sum(probs * jnp.log(probs + 1e-10))
        perp_ref[0, 0] = jnp.exp(ent)


def _vq_call(x3, embed, interpret=False):
    return pl.pallas_call(
        _vq_body,
        grid=(NSTEP,),
        in_specs=[
            pl.BlockSpec((BPS, C, HW), lambda b: (b, 0, 0)),
            pl.BlockSpec((NE, C), lambda b: (0, 0)),
        ],
        out_specs=[
            pl.BlockSpec((BPS, C, HW), lambda b: (b, 0, 0)),
            pl.BlockSpec(memory_space=pltpu.SMEM),
            pl.BlockSpec(memory_space=pltpu.SMEM),
        ],
        out_shape=[
            jax.ShapeDtypeStruct((NB, C, HW), jnp.float32),
            jax.ShapeDtypeStruct((1, 1), jnp.float32),
            jax.ShapeDtypeStruct((1, 1), jnp.float32),
        ],
        scratch_shapes=[
            pltpu.VMEM((NE, C), jnp.float32),
            pltpu.VMEM((NE, 2), jnp.float32),
            pltpu.VMEM((NE, 1), jnp.float32),
            pltpu.SMEM((1,), jnp.float32),
        ],
        interpret=interpret,
    )(x3, embed)


def kernel(inputs, embed):
    x3 = inputs.reshape(NB, C, HW)
    zq, loss, perp = _vq_call(x3, embed)
    z_q_out = zq.reshape(NB, C, 32, 32)
    kldiv_r = math.log(NE) * HW * jnp.ones((NB, 1), dtype=jnp.float32)
    return (z_q_out, loss[0, 0], kldiv_r, perp[0, 0])


# speedup vs baseline: 1.0964x; 1.0964x over previous
import math

import jax
import jax.numpy as jnp
from jax import lax
from jax.experimental import pallas as pl
from jax.experimental.pallas import tpu as pltpu

NB = 16
C = 64
HW = 1024
NE = 1024
N_TOK = NB * HW
BPS = 4
NSTEP = NB // BPS


def _quantize(x, emb, em2, e2):
    mm2 = lax.dot_general(em2, x, (((1,), (0,)), ((), ())),
                          preferred_element_type=jnp.float32)
    xn = jnp.sum(x * x, axis=0, keepdims=True)
    d2 = (xn + e2) + mm2
    mind = jnp.min(d2, axis=0, keepdims=True)
    eqm = d2 == mind
    m = jnp.where(eqm, 1.0, 0.0)
    zq = lax.dot_general(emb, m, (((0,), (0,)), ((), ())),
                         preferred_element_type=jnp.float32)
    diff = zq - x
    zq_st = x + diff
    s = jnp.sum(diff * diff)
    h = jnp.sum(m, axis=1, keepdims=True)
    return zq_st, s, h, eqm


def _exact_onehot(eqm):
    iota = lax.broadcasted_iota(jnp.int32, eqm.shape, 0)
    idxv = jnp.min(jnp.where(eqm, iota, NE), axis=0, keepdims=True)
    return (iota == idxv).astype(jnp.float32)


def _vq_body(x_ref, e_ref, zq_ref, loss_ref, perp_ref,
             em2_ref, e2_ref, hist_ref, acc_ref):
    b = pl.program_id(0)
    emb = e_ref[...]

    @pl.when(b == 0)
    def _init():
        acc_ref[0] = 0.0
        hist_ref[...] = jnp.zeros_like(hist_ref)
        e2_ref[...] = jnp.sum(emb * emb, axis=1, keepdims=True)
        em2_ref[...] = emb * (-2.0)

    em2 = em2_ref[...]
    e2 = e2_ref[...]

    res = [_quantize(x_ref[j], emb, em2, e2) for j in range(BPS)]
    for j, (zq_st, _, _, _) in enumerate(res):
        zq_ref[j] = zq_st
    s_all = res[0][1]
    h_all = res[0][2]
    for j in range(1, BPS):
        s_all = s_all + res[j][1]
        h_all = h_all + res[j][2]
    acc_ref[0] += s_all
    hist_ref[...] += h_all

    @pl.when(jnp.sum(h_all) > BPS * HW + 0.5)
    def _fix():
        for j in range(BPS):
            _, s1, h1, eqm = res[j]
            x_j = x_ref[j]
            onehot = _exact_onehot(eqm)
            zq2 = lax.dot_general(emb, onehot, (((0,), (0,)), ((), ())),
                                  preferred_element_type=jnp.float32)
            diff2 = zq2 - x_j
            zq_ref[j] = x_j + diff2
            acc_ref[0] += jnp.sum(diff2 * diff2) - s1
            hist_ref[...] += jnp.sum(onehot, axis=1, keepdims=True) - h1

    @pl.when(b == NSTEP - 1)
    def _fini():
        loss_ref[0, 0] = 1.25 * acc_ref[0] / (NB * C * HW)
        probs = hist_ref[...] * (1.0 / N_TOK)
        ent = -jnp.sum(probs * jnp.log(probs + 1e-10))
        perp_ref[0, 0] = jnp.exp(ent)


def _vq_call(x3, embed, interpret=False):
    return pl.pallas_call(
        _vq_body,
        grid=(NSTEP,),
        in_specs=[
            pl.BlockSpec((BPS, C, HW), lambda b: (b, 0, 0)),
            pl.BlockSpec((NE, C), lambda b: (0, 0)),
        ],
        out_specs=[
            pl.BlockSpec((BPS, C, HW), lambda b: (b, 0, 0)),
            pl.BlockSpec(memory_space=pltpu.SMEM),
            pl.BlockSpec(memory_space=pltpu.SMEM),
        ],
        out_shape=[
            jax.ShapeDtypeStruct((NB, C, HW), jnp.float32),
            jax.ShapeDtypeStruct((1, 1), jnp.float32),
            jax.ShapeDtypeStruct((1, 1), jnp.float32),
        ],
        scratch_shapes=[
            pltpu.VMEM((NE, C), jnp.float32),
            pltpu.VMEM((NE, 1), jnp.float32),
            pltpu.VMEM((NE, 1), jnp.float32),
            pltpu.SMEM((1,), jnp.float32),
        ],
        interpret=interpret,
    )(x3, embed)


def kernel(inputs, embed):
    x3 = inputs.reshape(NB, C, HW)
    zq, loss, perp = _vq_call(x3, embed)
    z_q_out = zq.reshape(NB, C, 32, 32)
    kldiv_r = math.log(NE) * HW * jnp.ones((NB, 1), dtype=jnp.float32)
    return (z_q_out, loss[0, 0], kldiv_r, perp[0, 0])
